# baseline (speedup 1.0000x reference)
import functools
import jax
import jax.numpy as jnp
from jax import lax
from jax.experimental import pallas as pl
from jax.experimental.pallas import tpu as pltpu
from jax.experimental.pallas import tpu_sc as plsc

K = 512
D = 256
BETA = 0.25
N = 27648

_TILE = 4608
_TPB = 13824 // _TILE

NW = 32
BPW = N // NW
CHUNK = 48
NCH = BPW // CHUNK
CPR = K // 16


def _vq_body(x_ref, w_ref, idx_ref, q_ref, loss_ref):
    b = pl.program_id(0)
    t = pl.program_id(1)
    xt = x_ref[...]
    w = w_ref[...]
    x_sq = jnp.sum(xt * xt, axis=1, keepdims=True)
    w_sq = jnp.sum(w * w, axis=1)
    mm = jax.lax.dot_general(xt, w, (((1,), (1,)), ((), ())),
                             preferred_element_type=jnp.float32)
    d = x_sq + w_sq - 2.0 * mm
    dmin = jnp.min(d, axis=1, keepdims=True)
    iota_f = jax.lax.broadcasted_iota(jnp.int32, d.shape, 1).astype(jnp.float32)
    idx = jnp.min(jnp.where(d == dmin, iota_f, float(K)),
                  axis=1, keepdims=True)
    enc = (iota_f == idx).astype(jnp.float32)
    idx_ref[...] = idx.astype(jnp.int32)
    q_ref[...] = jax.lax.dot_general(enc, w, (((1,), (0,)), ((), ())),
                                     preferred_element_type=jnp.float32)

    @pl.when((b == 0) & (t == 0))
    def _():
        loss_ref[...] = jnp.zeros((1, 1), jnp.float32)

    loss_ref[...] += jnp.sum(dmin)[None, None]


_mesh = plsc.VectorSubcoreMesh(core_axis_name="c", subcore_axis_name="s")


@functools.partial(
    pl.kernel, mesh=_mesh,
    compiler_params=pltpu.CompilerParams(needs_layout_passes=False),
    out_type=jax.ShapeDtypeStruct((N * K,), jnp.float32),
    scratch_types=[
        pltpu.VMEM((BPW,), jnp.int32),
        pltpu.VMEM((CHUNK * K,), jnp.float32),
        pltpu.VMEM((CHUNK * K,), jnp.float32),
        pltpu.SemaphoreType.DMA,
        pltpu.SemaphoreType.DMA,
    ],
)
def _sc_enc(idx_hbm, zeros_hbm, enc_hbm, idx_v, buf0, buf1, sem0, sem1):
    wid = lax.axis_index("s") * 2 + lax.axis_index("c")
    base = wid * BPW
    pltpu.sync_copy(idx_hbm.at[pl.ds(base, BPW)], idx_v)
    pltpu.sync_copy(zeros_hbm, buf0)
    pltpu.sync_copy(zeros_hbm, buf1)
    row_iota = lax.iota(jnp.int32, 16)
    ones16 = jnp.full((16,), 1.0, jnp.float32)
    zeros16 = jnp.zeros((16,), jnp.float32)
    bufs = (buf0, buf1)
    sems = (sem0, sem1)
    nrow = CHUNK // 16

    def scatter(buf, g, val):
        for j in range(nrow):
            col = idx_v[pl.ds(g * CHUNK + j * 16, 16)]
            flat = (row_iota + j * 16) * K + col
            plsc.store_scatter(buf, [flat], val)

    def start(buf, sem, g):
        pltpu.make_async_copy(
            buf, enc_hbm.at[pl.ds((base + g * CHUNK) * K, CHUNK * K)],
            sem).start()

    def wait(buf, sem, g):
        pltpu.make_async_copy(
            buf, enc_hbm.at[pl.ds((base + g * CHUNK) * K, CHUNK * K)],
            sem).wait()

    for s in range(2):
        scatter(bufs[s], s, ones16)
        start(bufs[s], sems[s], s)

    def pair_body(p, _):
        for s in range(2):
            g = 2 * p + s
            wait(bufs[s], sems[s], g - 2)
            scatter(bufs[s], g - 2, zeros16)
            scatter(bufs[s], g, ones16)
            start(bufs[s], sems[s], g)
        return _

    lax.fori_loop(1, NCH // 2, pair_body, 0)
    wait(buf0, sem0, NCH - 2)
    wait(buf1, sem1, NCH - 1)


def kernel(x, W):
    B, C, D1, D2, D3 = x.shape
    S = D1 * D2 * D3
    x_flat = jnp.transpose(x, (0, 2, 3, 4, 1)).reshape(N, D)
    idx_arr, quant, loss_sum = pl.pallas_call(
        _vq_body,
        grid=(B, _TPB),
        in_specs=[
            pl.BlockSpec((_TILE, D), lambda b, t: (b * _TPB + t, 0)),
            pl.BlockSpec((K, D), lambda b, t: (0, 0)),
        ],
        out_specs=[
            pl.BlockSpec((_TILE, 1), lambda b, t: (b * _TPB + t, 0)),
            pl.BlockSpec((_TILE, D), lambda b, t: (b * _TPB + t, 0)),
            pl.BlockSpec((1, 1), lambda b, t: (0, 0)),
        ],
        out_shape=[
            jax.ShapeDtypeStruct((N, 1), jnp.int32),
            jax.ShapeDtypeStruct((N, D), jnp.float32),
            jax.ShapeDtypeStruct((1, 1), jnp.float32),
        ],
    )(x_flat, W)
    zeros_chunk = jnp.zeros((CHUNK * K,), jnp.float32)
    enc = _sc_enc(idx_arr.reshape(N), zeros_chunk).reshape(N, K)
    mse = loss_sum[0, 0] / (N * D)
    e_latent = jnp.clip(mse, 0.0, 10.0)
    loss = e_latent + BETA * e_latent
    out = jnp.transpose(quant.reshape(B, D1, D2, D3, C), (0, 4, 1, 2, 3))
    return (loss, out, enc)


# speedup vs baseline: 3.5896x; 3.5896x over previous
import jax
import jax.numpy as jnp
from jax.experimental import pallas as pl
from jax.experimental.pallas import tpu as pltpu

K = 512
D = 256
BETA = 0.25

_TILE = 4608
_TPB = 13824 // _TILE


def _vq_body(x_ref, w_ref, enc_ref, q_ref, loss_ref):
    b = pl.program_id(0)
    t = pl.program_id(1)
    xt = x_ref[...]
    w = w_ref[...]
    x_sq = jnp.sum(xt * xt, axis=1, keepdims=True)
    w_sq = jnp.sum(w * w, axis=1)
    mm = jax.lax.dot_general(xt, w, (((1,), (1,)), ((), ())),
                             preferred_element_type=jnp.float32)
    d = x_sq + w_sq - 2.0 * mm
    dmin = jnp.min(d, axis=1, keepdims=True)
    iota_f = jax.lax.broadcasted_iota(jnp.int32, (1, K), 1).astype(jnp.float32)
    idx = jnp.min(jnp.where(d == dmin, iota_f, float(K)),
                  axis=1, keepdims=True)
    enc = (iota_f == idx).astype(jnp.float32)
    enc_ref[...] = enc
    q_ref[...] = jax.lax.dot_general(enc, w, (((1,), (0,)), ((), ())),
                                     preferred_element_type=jnp.float32)

    @pl.when((b == 0) & (t == 0))
    def _():
        loss_ref[...] = jnp.zeros((1, 1), jnp.float32)

    ones_row = jnp.ones((1, _TILE), jnp.float32)
    loss_ref[...] += jax.lax.dot_general(
        ones_row, dmin, (((1,), (0,)), ((), ())),
        preferred_element_type=jnp.float32)


def kernel(x, W):
    B, C, D1, D2, D3 = x.shape
    S = D1 * D2 * D3
    N = B * S
    x_flat = jnp.transpose(x, (0, 2, 3, 4, 1)).reshape(N, D)
    enc, quant, loss_sum = pl.pallas_call(
        _vq_body,
        grid=(B, _TPB),
        in_specs=[
            pl.BlockSpec((_TILE, D), lambda b, t: (b * _TPB + t, 0)),
            pl.BlockSpec((K, D), lambda b, t: (0, 0)),
        ],
        out_specs=[
            pl.BlockSpec((_TILE, K), lambda b, t: (b * _TPB + t, 0)),
            pl.BlockSpec((_TILE, D), lambda b, t: (b * _TPB + t, 0)),
            pl.BlockSpec((1, 1), lambda b, t: (0, 0)),
        ],
        out_shape=[
            jax.ShapeDtypeStruct((N, K), jnp.float32),
            jax.ShapeDtypeStruct((N, D), jnp.float32),
            jax.ShapeDtypeStruct((1, 1), jnp.float32),
        ],
    )(x_flat, W)
    mse = loss_sum[0, 0] / (N * D)
    e_latent = jnp.clip(mse, 0.0, 10.0)
    loss = e_latent + BETA * e_latent
    out = jnp.transpose(quant.reshape(B, D1, D2, D3, C), (0, 4, 1, 2, 3))
    return (loss, out, enc)
